# baseline (speedup 1.0000x reference)
import dataclasses
import functools

import numpy as np
import jax
import jax.numpy as jnp
from jax import lax
from jax.experimental import pallas as pl
from jax.experimental.pallas import tpu as pltpu
from jax.experimental.pallas import tpu_sc as plsc

NUM_LEVELS = 16
MIN_RES = 16
MAX_RES = 1024
LOG2_HASHMAP_SIZE = 18
FPL = 2
TABLE_SIZE = 2 ** LOG2_HASHMAP_SIZE
PVR = 2 ** (LOG2_HASHMAP_SIZE // 3)
_growth = np.exp((np.log(MAX_RES) - np.log(MIN_RES)) / (NUM_LEVELS - 1))
_SCALINGS = np.floor(MIN_RES * _growth ** np.arange(NUM_LEVELS)).astype(np.float32)

NCELL = NUM_LEVELS * PVR * PVR * PVR
NW = 32
SC_CH = 256
G = SC_CH // 16
OUTD = NUM_LEVELS * FPL
SLABS = NUM_LEVELS * PVR
SLAB_PER_W = SLABS // NW
SLAB_F = PVR * PVR
_CORNERS = [(dx, dy, dz) for dx in (0, 1) for dy in (0, 1) for dz in (0, 1)]


def _mesh_and_params():
    mesh = plsc.VectorSubcoreMesh(core_axis_name="c", subcore_axis_name="s")
    cp = pltpu.CompilerParams()
    fields = pltpu.CompilerParams.__dataclass_fields__
    if "needs_layout_passes" in fields:
        cp = dataclasses.replace(cp, needs_layout_passes=False)
    if "use_tc_tiling_on_sc" in fields:
        cp = dataclasses.replace(cp, use_tc_tiling_on_sc=False)
    return mesh, cp


def _build_cells(t0, t1):
    mesh, cp = _mesh_and_params()

    @functools.partial(
        pl.kernel,
        out_type=jax.ShapeDtypeStruct((NCELL, 8), jnp.int32),
        mesh=mesh,
        compiler_params=cp,
        scratch_types=[
            pltpu.VMEM((2, 4, SLAB_F), jnp.float32),
            pltpu.VMEM((SLAB_F, 8), jnp.int32),
            pltpu.SemaphoreType.DMA,
            pltpu.SemaphoreType.DMA,
        ],
    )
    def build(t0_h, t1_h, cell_h, inbuf, cellbuf, sem0, sem1):
        wid = lax.axis_index("s") * 2 + lax.axis_index("c")
        iota = lax.iota(jnp.int32, 16)
        lane15 = iota == 15
        sems = (sem0, sem1)
        ths = (t0_h, t1_h)

        def slab_starts(si):
            slab = wid * SLAB_PER_W + si
            x = slab & (PVR - 1)
            slab1 = jnp.where(x == PVR - 1, slab - (PVR - 1), slab + 1)
            return slab * SLAB_F, slab1 * SLAB_F

        def stage(si, b):
            s0, s1 = slab_starts(si)
            cps = []
            for j in (0, 1):
                cps.append(pltpu.async_copy(ths[j].at[pl.ds(s0, SLAB_F)], inbuf.at[b, 2 * j], sems[b]))
                cps.append(pltpu.async_copy(ths[j].at[pl.ds(s1, SLAB_F)], inbuf.at[b, 2 * j + 1], sems[b]))
            return cps

        def wait_stage(si, b):
            s0, s1 = slab_starts(si)
            for j in (0, 1):
                pltpu.make_async_copy(ths[j].at[pl.ds(s0, SLAB_F)], inbuf.at[b, 2 * j], sems[b]).wait()
                pltpu.make_async_copy(ths[j].at[pl.ds(s1, SLAB_F)], inbuf.at[b, 2 * j + 1], sems[b]).wait()

        def build_slab(si, b):
            wait_stage(si, b)

            @pl.loop(0, PVR)
            def _(y):
                ybase = (y * PVR, ((y + 1) & (PVR - 1)) * PVR)
                for zg in range(4):
                    z0 = zg * 16
                    rows = y * PVR + z0 + iota
                    vals = []
                    for c, (dx, dy, dz) in enumerate(_CORNERS):
                        fv = []
                        for j in (0, 1):
                            bsrc = inbuf.at[b, 2 * j + dx]
                            v = bsrc[pl.ds(ybase[dy] + z0 + dz, 16)]
                            if dz == 1 and zg == 3:
                                w = plsc.load_gather(bsrc, [jnp.full((16,), ybase[dy], jnp.int32)])
                                v = jnp.where(lane15, w, v)
                            fv.append(v)
                        packed = plsc.bitcast(
                            plsc.pack(fv[0], fv[1], format=plsc.PackFormat.INTERLEAVED),
                            jnp.int32)
                        vals.append(packed)
                    for c, v in enumerate(vals):
                        colv = jnp.full((16,), c, jnp.int32)
                        plsc.store_scatter(cellbuf, [rows, colv], v)

            slab = wid * SLAB_PER_W + si
            pltpu.sync_copy(cellbuf, cell_h.at[pl.ds(slab * SLAB_F, SLAB_F)])

        stage(0, 0)

        @pl.loop(0, SLAB_PER_W, step=2)
        def _(si):
            stage(si + 1, 1)
            build_slab(si, 0)

            @pl.when(si + 2 < SLAB_PER_W)
            def _():
                stage(si + 2, 0)

            build_slab(si + 1, 1)

    return build(t0, t1)


def _interp(xs, ys, zs, cell, scal, n):
    mesh, cp = _mesh_and_params()
    ppw = n // NW
    steps = (ppw // SC_CH) * NUM_LEVELS

    @functools.partial(
        pl.kernel,
        out_type=jax.ShapeDtypeStruct((n, OUTD), jnp.float32),
        mesh=mesh,
        compiler_params=cp,
        scratch_types=[
            pltpu.VMEM((ppw,), jnp.float32),
            pltpu.VMEM((ppw,), jnp.float32),
            pltpu.VMEM((ppw,), jnp.float32),
            pltpu.VMEM((2, SC_CH), jnp.int32),
            pltpu.VMEM((2, SC_CH, 8), jnp.int32),
            pltpu.VMEM((2, SC_CH), jnp.float32),
            pltpu.VMEM((2, SC_CH), jnp.float32),
            pltpu.VMEM((2, SC_CH), jnp.float32),
            pltpu.VMEM((SC_CH, OUTD), jnp.float32),
            pltpu.VMEM((NUM_LEVELS, 16), jnp.float32),
            pltpu.SemaphoreType.DMA,
            pltpu.SemaphoreType.DMA,
        ],
    )
    def run(xs_h, ys_h, zs_h, cell_h, scal_h, out_h,
            xv, yv, zv, idxv, dstv, oxv, oyv, ozv, outv, scal_v, sem0, sem1):
        wid = lax.axis_index("s") * 2 + lax.axis_index("c")
        base = wid * ppw
        pltpu.sync_copy(scal_h, scal_v)
        pltpu.sync_copy(xs_h.at[pl.ds(base, ppw)], xv)
        pltpu.sync_copy(ys_h.at[pl.ds(base, ppw)], yv)
        pltpu.sync_copy(zs_h.at[pl.ds(base, ppw)], zv)
        iota = lax.iota(jnp.int32, 16)
        sems = (sem0, sem1)

        def issue(s, b):
            l = s & (NUM_LEVELS - 1)
            p0 = (s >> 4) * SC_CH
            scale = scal_v[l]
            lofs = l * (PVR * PVR * PVR)
            for g in range(G):
                sl = pl.ds(p0 + g * 16, 16)
                sx = xv[sl] * scale
                sy = yv[sl] * scale
                sz = zv[sl] * scale
                fx = sx.astype(jnp.int32)
                fy = sy.astype(jnp.int32)
                fz = sz.astype(jnp.int32)
                gsl = pl.ds(g * 16, 16)
                oxv[b, gsl] = sx - fx.astype(jnp.float32)
                oyv[b, gsl] = sy - fy.astype(jnp.float32)
                ozv[b, gsl] = sz - fz.astype(jnp.float32)
                mx = fx & (PVR - 1)
                my = fy & (PVR - 1)
                mz = fz & (PVR - 1)
                idxv[b, gsl] = (((mx << 6) | my) << 6) | mz | lofs
            for h in (0, 1):
                pltpu.async_copy(cell_h.at[idxv.at[b, pl.ds(h * 128, 128)]],
                                 dstv.at[b, pl.ds(h * 128, 128)], sems[b])

        def consume(s, b):
            l = s & (NUM_LEVELS - 1)
            sc = s >> 4
            for h in (0, 1):
                pltpu.make_async_copy(cell_h.at[idxv.at[b, pl.ds(h * 128, 128)]],
                                      dstv.at[b, pl.ds(h * 128, 128)], sems[b]).wait()
            d = dstv.at[b]
            for g in range(G):
                gsl = pl.ds(g * 16, 16)
                ox = oxv[b, gsl]
                oy = oyv[b, gsl]
                oz = ozv[b, gsl]
                rows = iota + g * 16
                fw = {}
                for dx in (0, 1):
                    for dy in (0, 1):
                        for dz in (0, 1):
                            c = dx * 4 + dy * 2 + dz
                            col = jnp.full((16,), c, jnp.int32)
                            fw[(dx, dy, dz)] = plsc.load_gather(d, [rows, col])

                oxb = plsc.pack(ox, ox, format=plsc.PackFormat.INTERLEAVED)
                a = {}
                for dy in (0, 1):
                    for dz in (0, 1):
                        ac = plsc.bitcast(fw[(1, dy, dz)], jnp.bfloat16)
                        af = plsc.bitcast(fw[(0, dy, dz)], jnp.bfloat16)
                        v = af + oxb * (ac - af)
                        w = plsc.bitcast(v, jnp.int32)
                        a[(dy, dz, 0)] = plsc.bitcast(w << 16, jnp.float32)
                        a[(dy, dz, 1)] = plsc.bitcast(w & jnp.int32(-65536), jnp.float32)

                def lerp(a_c, a_f, w):
                    return a_f + w * (a_c - a_f)

                for j in range(FPL):
                    w1 = lerp(a[(1, 1, j)], a[(0, 1, j)], oy)
                    w0 = lerp(a[(1, 0, j)], a[(0, 0, j)], oy)
                    res = lerp(w1, w0, oz)
                    colv = jnp.full((16,), 2 * l + j, jnp.int32)
                    plsc.store_scatter(outv, [rows, colv], res)

            @pl.when(l == NUM_LEVELS - 1)
            def _():
                pltpu.sync_copy(outv, out_h.at[pl.ds(base + sc * SC_CH, SC_CH)])

        issue(0, 0)

        @pl.loop(0, steps, step=2)
        def _(s):
            issue(s + 1, 1)
            consume(s, 0)

            @pl.when(s + 2 < steps)
            def _():
                issue(s + 2, 0)

            consume(s + 1, 1)

    return run(xs, ys, zs, cell, scal)


def kernel(in_tensor, hash_table):
    n = in_tensor.shape[0]
    xs = in_tensor[:, 0]
    ys = in_tensor[:, 1]
    zs = in_tensor[:, 2]
    t0 = hash_table[:, 0]
    t1 = hash_table[:, 1]
    scal = jnp.broadcast_to(jnp.asarray(_SCALINGS)[:, None], (NUM_LEVELS, 16))
    cell = _build_cells(t0, t1)
    return _interp(xs, ys, zs, cell, scal, n)


# speedup vs baseline: 1.0840x; 1.0840x over previous
import dataclasses
import functools

import numpy as np
import jax
import jax.numpy as jnp
from jax import lax
from jax.experimental import pallas as pl
from jax.experimental.pallas import tpu as pltpu
from jax.experimental.pallas import tpu_sc as plsc

NUM_LEVELS = 16
MIN_RES = 16
MAX_RES = 1024
LOG2_HASHMAP_SIZE = 18
FPL = 2
TABLE_SIZE = 2 ** LOG2_HASHMAP_SIZE
PVR = 2 ** (LOG2_HASHMAP_SIZE // 3)
_growth = np.exp((np.log(MAX_RES) - np.log(MIN_RES)) / (NUM_LEVELS - 1))
_SCALINGS = np.floor(MIN_RES * _growth ** np.arange(NUM_LEVELS)).astype(np.float32)

NCELL = NUM_LEVELS * PVR * PVR * PVR
NW = 32
SC_CH = 128
G = SC_CH // 16
OUTD = NUM_LEVELS * FPL
SLABS = NUM_LEVELS * PVR
SLAB_PER_W = SLABS // NW
SLAB_F = PVR * PVR
_CORNERS = [(dx, dy, dz) for dx in (0, 1) for dy in (0, 1) for dz in (0, 1)]


def _mesh_and_params():
    mesh = plsc.VectorSubcoreMesh(core_axis_name="c", subcore_axis_name="s")
    cp = pltpu.CompilerParams()
    fields = pltpu.CompilerParams.__dataclass_fields__
    if "needs_layout_passes" in fields:
        cp = dataclasses.replace(cp, needs_layout_passes=False)
    if "use_tc_tiling_on_sc" in fields:
        cp = dataclasses.replace(cp, use_tc_tiling_on_sc=False)
    return mesh, cp


def _build_cells(t0, t1):
    mesh, cp = _mesh_and_params()

    @functools.partial(
        pl.kernel,
        out_type=jax.ShapeDtypeStruct((NCELL, 8), jnp.int32),
        mesh=mesh,
        compiler_params=cp,
        scratch_types=[
            pltpu.VMEM((2, 4, SLAB_F), jnp.float32),
            pltpu.VMEM((SLAB_F, 8), jnp.int32),
            pltpu.SemaphoreType.DMA,
            pltpu.SemaphoreType.DMA,
        ],
    )
    def build(t0_h, t1_h, cell_h, inbuf, cellbuf, sem0, sem1):
        wid = lax.axis_index("s") * 2 + lax.axis_index("c")
        iota = lax.iota(jnp.int32, 16)
        lane15 = iota == 15
        sems = (sem0, sem1)
        ths = (t0_h, t1_h)

        def slab_starts(si):
            slab = wid * SLAB_PER_W + si
            x = slab & (PVR - 1)
            slab1 = jnp.where(x == PVR - 1, slab - (PVR - 1), slab + 1)
            return slab * SLAB_F, slab1 * SLAB_F

        def stage(si, b):
            s0, s1 = slab_starts(si)
            cps = []
            for j in (0, 1):
                cps.append(pltpu.async_copy(ths[j].at[pl.ds(s0, SLAB_F)], inbuf.at[b, 2 * j], sems[b]))
                cps.append(pltpu.async_copy(ths[j].at[pl.ds(s1, SLAB_F)], inbuf.at[b, 2 * j + 1], sems[b]))
            return cps

        def wait_stage(si, b):
            s0, s1 = slab_starts(si)
            for j in (0, 1):
                pltpu.make_async_copy(ths[j].at[pl.ds(s0, SLAB_F)], inbuf.at[b, 2 * j], sems[b]).wait()
                pltpu.make_async_copy(ths[j].at[pl.ds(s1, SLAB_F)], inbuf.at[b, 2 * j + 1], sems[b]).wait()

        def build_slab(si, b):
            wait_stage(si, b)

            @pl.loop(0, PVR)
            def _(y):
                ybase = (y * PVR, ((y + 1) & (PVR - 1)) * PVR)
                for zg in range(4):
                    z0 = zg * 16
                    rows = y * PVR + z0 + iota
                    vals = []
                    for c, (dx, dy, dz) in enumerate(_CORNERS):
                        fv = []
                        for j in (0, 1):
                            bsrc = inbuf.at[b, 2 * j + dx]
                            v = bsrc[pl.ds(ybase[dy] + z0 + dz, 16)]
                            if dz == 1 and zg == 3:
                                w = plsc.load_gather(bsrc, [jnp.full((16,), ybase[dy], jnp.int32)])
                                v = jnp.where(lane15, w, v)
                            fv.append(v)
                        packed = plsc.bitcast(
                            plsc.pack(fv[0], fv[1], format=plsc.PackFormat.INTERLEAVED),
                            jnp.int32)
                        vals.append(packed)
                    for c, v in enumerate(vals):
                        colv = jnp.full((16,), c, jnp.int32)
                        plsc.store_scatter(cellbuf, [rows, colv], v)

            slab = wid * SLAB_PER_W + si
            pltpu.sync_copy(cellbuf, cell_h.at[pl.ds(slab * SLAB_F, SLAB_F)])

        stage(0, 0)

        @pl.loop(0, SLAB_PER_W, step=2)
        def _(si):
            stage(si + 1, 1)
            build_slab(si, 0)

            @pl.when(si + 2 < SLAB_PER_W)
            def _():
                stage(si + 2, 0)

            build_slab(si + 1, 1)

    return build(t0, t1)


def _interp(xs, ys, zs, cell, scal, n):
    mesh, cp = _mesh_and_params()
    ppw = n // NW
    steps = (ppw // SC_CH) * NUM_LEVELS

    @functools.partial(
        pl.kernel,
        out_type=jax.ShapeDtypeStruct((n, OUTD), jnp.float32),
        mesh=mesh,
        compiler_params=cp,
        scratch_types=[
            pltpu.VMEM((ppw,), jnp.float32),
            pltpu.VMEM((ppw,), jnp.float32),
            pltpu.VMEM((ppw,), jnp.float32),
            pltpu.VMEM((2, SC_CH), jnp.int32),
            pltpu.VMEM((2, SC_CH, 8), jnp.int32),
            pltpu.VMEM((2, SC_CH), jnp.float32),
            pltpu.VMEM((2, SC_CH), jnp.float32),
            pltpu.VMEM((2, SC_CH), jnp.float32),
            pltpu.VMEM((SC_CH, OUTD), jnp.float32),
            pltpu.VMEM((NUM_LEVELS, 16), jnp.float32),
            pltpu.SemaphoreType.DMA,
            pltpu.SemaphoreType.DMA,
        ],
    )
    def run(xs_h, ys_h, zs_h, cell_h, scal_h, out_h,
            xv, yv, zv, idxv, dstv, oxv, oyv, ozv, outv, scal_v, sem0, sem1):
        wid = lax.axis_index("s") * 2 + lax.axis_index("c")
        base = wid * ppw
        pltpu.sync_copy(scal_h, scal_v)
        pltpu.sync_copy(xs_h.at[pl.ds(base, ppw)], xv)
        pltpu.sync_copy(ys_h.at[pl.ds(base, ppw)], yv)
        pltpu.sync_copy(zs_h.at[pl.ds(base, ppw)], zv)
        iota = lax.iota(jnp.int32, 16)
        sems = (sem0, sem1)

        def issue(s, b):
            l = s & (NUM_LEVELS - 1)
            p0 = (s >> 4) * SC_CH
            scale = scal_v[l]
            lofs = l * (PVR * PVR * PVR)
            for g in range(G):
                sl = pl.ds(p0 + g * 16, 16)
                sx = xv[sl] * scale
                sy = yv[sl] * scale
                sz = zv[sl] * scale
                fx = sx.astype(jnp.int32)
                fy = sy.astype(jnp.int32)
                fz = sz.astype(jnp.int32)
                gsl = pl.ds(g * 16, 16)
                oxv[b, gsl] = sx - fx.astype(jnp.float32)
                oyv[b, gsl] = sy - fy.astype(jnp.float32)
                ozv[b, gsl] = sz - fz.astype(jnp.float32)
                mx = fx & (PVR - 1)
                my = fy & (PVR - 1)
                mz = fz & (PVR - 1)
                idxv[b, gsl] = (((mx << 6) | my) << 6) | mz | lofs
            return pltpu.async_copy(cell_h.at[idxv.at[b]], dstv.at[b], sems[b])

        def consume(s, b):
            l = s & (NUM_LEVELS - 1)
            sc = s >> 4
            pltpu.make_async_copy(cell_h.at[idxv.at[b]], dstv.at[b], sems[b]).wait()
            d = dstv.at[b]
            for g in range(G):
                gsl = pl.ds(g * 16, 16)
                ox = oxv[b, gsl]
                oy = oyv[b, gsl]
                oz = ozv[b, gsl]
                rows = iota + g * 16
                fw = {}
                for dx in (0, 1):
                    for dy in (0, 1):
                        for dz in (0, 1):
                            c = dx * 4 + dy * 2 + dz
                            col = jnp.full((16,), c, jnp.int32)
                            fw[(dx, dy, dz)] = plsc.load_gather(d, [rows, col])

                oxb = plsc.pack(ox, ox, format=plsc.PackFormat.INTERLEAVED)
                a = {}
                for dy in (0, 1):
                    for dz in (0, 1):
                        ac = plsc.bitcast(fw[(1, dy, dz)], jnp.bfloat16)
                        af = plsc.bitcast(fw[(0, dy, dz)], jnp.bfloat16)
                        v = af + oxb * (ac - af)
                        w = plsc.bitcast(v, jnp.int32)
                        a[(dy, dz, 0)] = plsc.bitcast(w << 16, jnp.float32)
                        a[(dy, dz, 1)] = plsc.bitcast(w & jnp.int32(-65536), jnp.float32)

                def lerp(a_c, a_f, w):
                    return a_f + w * (a_c - a_f)

                for j in range(FPL):
                    w1 = lerp(a[(1, 1, j)], a[(0, 1, j)], oy)
                    w0 = lerp(a[(1, 0, j)], a[(0, 0, j)], oy)
                    res = lerp(w1, w0, oz)
                    colv = jnp.full((16,), 2 * l + j, jnp.int32)
                    plsc.store_scatter(outv, [rows, colv], res)

            @pl.when(l == NUM_LEVELS - 1)
            def _():
                pltpu.sync_copy(outv, out_h.at[pl.ds(base + sc * SC_CH, SC_CH)])

        issue(0, 0)

        @pl.loop(0, steps, step=2)
        def _(s):
            issue(s + 1, 1)
            consume(s, 0)

            @pl.when(s + 2 < steps)
            def _():
                issue(s + 2, 0)

            consume(s + 1, 1)

    return run(xs, ys, zs, cell, scal)


def kernel(in_tensor, hash_table):
    n = in_tensor.shape[0]
    xs = in_tensor[:, 0]
    ys = in_tensor[:, 1]
    zs = in_tensor[:, 2]
    t0 = hash_table[:, 0]
    t1 = hash_table[:, 1]
    scal = jnp.broadcast_to(jnp.asarray(_SCALINGS)[:, None], (NUM_LEVELS, 16))
    cell = _build_cells(t0, t1)
    return _interp(xs, ys, zs, cell, scal, n)
